# baseline (speedup 1.0000x reference)
import functools

import jax
import jax.numpy as jnp
from jax import lax
from jax.experimental import pallas as pl
from jax.experimental.pallas import tpu as pltpu
from jax.experimental.pallas import tpu_sc as plsc

D_MODEL = 1024
T = 8192

_NC = 2
_NS = 16
_NW = _NC * _NS
_BPW = T // _NW
_C = 16
_NCHUNK = _BPW // _C


@functools.partial(
    pl.kernel,
    mesh=plsc.VectorSubcoreMesh(core_axis_name="c", subcore_axis_name="s"),
    out_type=jax.ShapeDtypeStruct((1, T, D_MODEL), jnp.float32),
    scratch_types=[
        pltpu.VMEM((_BPW,), jnp.int32),
        pltpu.VMEM((_C, D_MODEL), jnp.float32),
        pltpu.VMEM((_C, D_MODEL), jnp.float32),
        pltpu.VMEM_SHARED((_NS, 2, _C, D_MODEL), jnp.float32),
        pltpu.SemaphoreType.DMA,
        pltpu.SemaphoreType.DMA,
        pltpu.SemaphoreType.DMA,
        pltpu.SemaphoreType.DMA,
        pltpu.SemaphoreType.DMA,
        pltpu.SemaphoreType.DMA,
    ],
)
def _gather_rows(types_hbm, emb_hbm, out_hbm, idx_v, buf0, buf1, spstage,
                 g0, g1, w0, w1, d0, d1):
    sid = lax.axis_index("s")
    wid = sid * _NC + lax.axis_index("c")
    base = wid * _BPW
    pltpu.sync_copy(types_hbm.at[pl.ds(base, _BPW)], idx_v)
    bufs = (buf0, buf1)
    gsem = (g0, g1)
    wsem = (w0, w1)
    dsem = (d0, d1)

    def is_b(c):
        return c % 2 == 1

    def slot(c):
        return (c // 2) % 2

    gh = [None] * _NCHUNK
    bh = [None] * _NCHUNK
    dh = [None] * _NCHUNK
    gh[0] = pltpu.async_copy(
        emb_hbm.at[idx_v.at[pl.ds(0, _C)]], bufs[0], gsem[0])
    for c in range(_NCHUNK):
        b = c % 2
        if c + 1 < _NCHUNK:
            if c >= 1:
                bh[c - 1].wait()
                if is_b(c - 1):
                    dh[c - 1] = pltpu.async_copy(
                        spstage.at[sid, slot(c - 1)],
                        out_hbm.at[0, pl.ds(base + (c - 1) * _C, _C)],
                        dsem[slot(c - 1)])
            gh[c + 1] = pltpu.async_copy(
                emb_hbm.at[idx_v.at[pl.ds((c + 1) * _C, _C)]],
                bufs[(c + 1) % 2], gsem[(c + 1) % 2])
        gh[c].wait()
        if is_b(c):
            s = slot(c)
            if c - 4 >= 0 and is_b(c - 4):
                dh[c - 4].wait()
            bh[c] = pltpu.async_copy(bufs[b], spstage.at[sid, s], wsem[b])
        else:
            bh[c] = pltpu.async_copy(
                bufs[b], out_hbm.at[0, pl.ds(base + c * _C, _C)], wsem[b])
    bh[_NCHUNK - 2].wait()
    bh[_NCHUNK - 1].wait()
    for c in (_NCHUNK - 2, _NCHUNK - 1):
        if is_b(c) and dh[c] is None:
            dh[c] = pltpu.async_copy(
                spstage.at[sid, slot(c)],
                out_hbm.at[0, pl.ds(base + c * _C, _C)],
                dsem[slot(c)])
    for c in range(_NCHUNK):
        if is_b(c) and c >= _NCHUNK - 4:
            dh[c].wait()


def kernel(types, emb):
    return _gather_rows(types.astype(jnp.int32), emb)


# speedup vs baseline: 1.0139x; 1.0139x over previous
import functools

import jax
import jax.numpy as jnp
from jax import lax
from jax.experimental import pallas as pl
from jax.experimental.pallas import tpu as pltpu
from jax.experimental.pallas import tpu_sc as plsc

D_MODEL = 1024
T = 8192

_NC = 2
_NS = 16
_NW = _NC * _NS
_BPW = T // _NW
_C = 32
_NCHUNK = _BPW // _C


@functools.partial(
    pl.kernel,
    mesh=plsc.VectorSubcoreMesh(core_axis_name="c", subcore_axis_name="s"),
    out_type=jax.ShapeDtypeStruct((1, T, D_MODEL), jnp.float32),
    scratch_types=[
        pltpu.VMEM((_BPW,), jnp.int32),
        pltpu.VMEM((_C, D_MODEL), jnp.float32),
        pltpu.VMEM((_C, D_MODEL), jnp.float32),
        pltpu.SemaphoreType.DMA,
        pltpu.SemaphoreType.DMA,
        pltpu.SemaphoreType.DMA,
        pltpu.SemaphoreType.DMA,
    ],
)
def _gather_rows(types_hbm, emb_hbm, out_hbm, idx_v, buf0, buf1,
                 g0, g1, w0, w1):
    wid = lax.axis_index("s") * _NC + lax.axis_index("c")
    base = wid * _BPW
    bufs = (buf0, buf1)
    gsem = (g0, g1)
    wsem = (w0, w1)
    pltpu.sync_copy(types_hbm.at[pl.ds(base, _C)], idx_v.at[pl.ds(0, _C)])
    gh = [None] * _NCHUNK
    wh = [None] * _NCHUNK
    gh[0] = pltpu.async_copy(
        emb_hbm.at[idx_v.at[pl.ds(0, _C)]], bufs[0], gsem[0])
    pltpu.sync_copy(types_hbm.at[pl.ds(base + _C, _BPW - _C)],
                    idx_v.at[pl.ds(_C, _BPW - _C)])
    for c in range(_NCHUNK):
        b = c % 2
        if c + 1 < _NCHUNK:
            nb = (c + 1) % 2
            if c >= 1:
                wh[c - 1].wait()
            gh[c + 1] = pltpu.async_copy(
                emb_hbm.at[idx_v.at[pl.ds((c + 1) * _C, _C)]],
                bufs[nb], gsem[nb])
        gh[c].wait()
        wh[c] = pltpu.async_copy(
            bufs[b], out_hbm.at[0, pl.ds(base + c * _C, _C)], wsem[b])
    wh[_NCHUNK - 2].wait()
    wh[_NCHUNK - 1].wait()


def kernel(types, emb):
    return _gather_rows(types.astype(jnp.int32), emb)
